# baseline (speedup 1.0000x reference)
import functools

import jax
import jax.numpy as jnp
from jax import lax
from jax.experimental import pallas as pl
import jax.experimental.pallas.tpu as pltpu
from jax.experimental.pallas import tpu_sc as plsc

_TOP_K = 8
_NEG = -1e30
_NW = 32
_L = 16
_ROWS_PER_W = 2


def _tc_body(facts_ref, beta_ref, al_ref, rs_ref, W_ref, act_ref, proj_ref):
    facts = facts_ref[...]
    B = facts.shape[0]
    beta = beta_ref[...]
    mask = jnp.where(beta > 0.0, 1.0, 0.0)
    mask_bf = mask.astype(jnp.bfloat16)

    dn = (((1,), (1,)), ((), ()))
    mm_bf = functools.partial(jax.lax.dot_general, dimension_numbers=dn,
                              preferred_element_type=jnp.float32)

    f_hi = facts.astype(jnp.bfloat16)
    r1 = facts - f_hi.astype(jnp.float32)
    f_lo = r1.astype(jnp.bfloat16)
    f_lo2 = (r1 - f_lo.astype(jnp.float32)).astype(jnp.bfloat16)

    log_f = jnp.log(jnp.maximum(facts, 1e-30))
    log_1mf = jnp.log(jnp.maximum(1.0 - facts, 1e-30))
    big_lhs = jnp.concatenate(
        [log_f.astype(jnp.bfloat16), log_1mf.astype(jnp.bfloat16),
         f_hi, f_lo, f_lo2], axis=0)
    big = mm_bf(big_lhs, mask_bf)
    prods = jnp.exp(big[:2 * B])
    and_agg = prods[:B]
    or_agg = 1.0 - prods[B:]
    s_sum = big[2 * B:3 * B] + big[3 * B:4 * B] + big[4 * B:]
    cnt = jnp.sum(mask, axis=1)[None, :] + 1e-08
    k_of_n = s_sum / cnt

    w = jax.nn.softmax(al_ref[...].T, axis=0)
    mixed = (w[0][None, :] * and_agg + w[1][None, :] * or_agg
             + w[2][None, :] * k_of_n + w[3][None, :] * (1.0 - k_of_n))
    act_ref[...] = mixed * jax.nn.sigmoid(rs_ref[...])

    W_f32 = W_ref[...]
    w_hi = W_f32.astype(jnp.bfloat16)
    w_lo = (W_f32 - w_hi.astype(jnp.float32)).astype(jnp.bfloat16)
    R = W_f32.shape[0]
    P = mm_bf(jnp.concatenate([f_hi, f_lo], axis=0),
              jnp.concatenate([w_hi, w_lo], axis=0))
    proj_ref[...] = (P[:B, :R] + P[:B, R:]) + (P[B:, :R] + P[B:, R:])


def _rsqrt16(v):
    i = plsc.bitcast(v, jnp.int32)
    i = jnp.int32(0x5F3759DF) - lax.shift_right_arithmetic(i, jnp.int32(1))
    y = plsc.bitcast(i, jnp.float32)
    half = v * 0.5
    for _ in range(3):
        y = y * (1.5 - half * y * y)
    return y


def _sc_body(act_hbm, proj_hbm, gamma_hbm, lnb_hbm, out_hbm,
             act_v, proj_v, out_v, gamma_v, lnb_v):
    R = act_hbm.shape[1]
    nch = R // _L
    wid = lax.axis_index("s") * 2 + lax.axis_index("c")
    base = wid * _ROWS_PER_W
    pltpu.sync_copy(act_hbm.at[pl.ds(base, _ROWS_PER_W), :], act_v)
    pltpu.sync_copy(proj_hbm.at[pl.ds(base, _ROWS_PER_W), :], proj_v)
    pltpu.sync_copy(gamma_hbm, gamma_v)
    pltpu.sync_copy(lnb_hbm, lnb_v)

    for r in range(_ROWS_PER_W):
        tops = [jnp.full((_L,), _NEG, jnp.float32) for _ in range(_TOP_K)]
        for c in range(nch):
            x = act_v[r, pl.ds(c * _L, _L)]
            for j in range(_TOP_K):
                hi = jnp.maximum(tops[j], x)
                x = jnp.minimum(tops[j], x)
                tops[j] = hi
        t = jnp.float32(0.0)
        for it in range(_TOP_K):
            m = jnp.max(tops[0])
            for j in range(1, _TOP_K):
                m = jnp.maximum(m, jnp.max(tops[j]))
            t = m
            if it < _TOP_K - 1:
                removed = jnp.int32(0)
                for j in range(_TOP_K):
                    eq = tops[j] == m
                    eqi = eq.astype(jnp.int32)
                    first = jnp.logical_and(eq, lax.cumsum(eqi) == 1)
                    any_eq = jnp.max(eqi) > 0
                    take = jnp.logical_and(any_eq, removed == 0)
                    sel = jnp.logical_and(first, jnp.broadcast_to(take, (_L,)))
                    tops[j] = jnp.where(sel, _NEG, tops[j])
                    removed = removed + take.astype(jnp.int32)
        gt_count = jnp.int32(0)
        for c in range(nch):
            x = act_v[r, pl.ds(c * _L, _L)]
            gt_count = gt_count + jnp.sum((x > t).astype(jnp.int32))
        need = _TOP_K - gt_count
        run_eq = jnp.int32(0)
        s1 = jnp.float32(0.0)
        s2 = jnp.float32(0.0)
        for c in range(nch):
            x = act_v[r, pl.ds(c * _L, _L)]
            gt = x > t
            eq = x == t
            eqi = eq.astype(jnp.int32)
            rank = lax.cumsum(eqi) - 1 + run_eq
            gate = jnp.logical_or(gt, jnp.logical_and(eq, rank < need))
            y = proj_v[r, pl.ds(c * _L, _L)] + jnp.where(gate, x, 0.0)
            out_v[r, pl.ds(c * _L, _L)] = y
            s1 = s1 + jnp.sum(y)
            s2 = s2 + jnp.sum(y * y)
            run_eq = run_eq + jnp.sum(eqi)
        inv_r = jnp.float32(1.0 / R)
        mu = s1 * inv_r
        var = s2 * inv_r - mu * mu
        rsq = _rsqrt16(jnp.broadcast_to(var + 1e-05, (_L,)))
        muv = jnp.broadcast_to(mu, (_L,))
        for c in range(nch):
            y = out_v[r, pl.ds(c * _L, _L)]
            g = gamma_v[pl.ds(c * _L, _L)]
            b = lnb_v[pl.ds(c * _L, _L)]
            out_v[r, pl.ds(c * _L, _L)] = (y - muv) * rsq * g + b

    pltpu.sync_copy(out_v, out_hbm.at[pl.ds(base, _ROWS_PER_W), :])


def kernel(facts, beta, aggregator_logits, rule_strength_raw, W, gamma,
           ln_beta):
    B, _ = facts.shape
    R, _ = beta.shape
    act, proj = pl.pallas_call(
        _tc_body,
        out_shape=(jax.ShapeDtypeStruct((B, R), jnp.float32),
                   jax.ShapeDtypeStruct((B, R), jnp.float32)),
    )(facts, beta, aggregator_logits, rule_strength_raw[None, :], W)

    mesh = plsc.VectorSubcoreMesh(core_axis_name="c", subcore_axis_name="s")
    sc = pl.kernel(
        _sc_body,
        mesh=mesh,
        compiler_params=pltpu.CompilerParams(needs_layout_passes=False),
        out_type=jax.ShapeDtypeStruct((B, R), jnp.float32),
        scratch_types=[
            pltpu.VMEM((_ROWS_PER_W, R), jnp.float32),
            pltpu.VMEM((_ROWS_PER_W, R), jnp.float32),
            pltpu.VMEM((_ROWS_PER_W, R), jnp.float32),
            pltpu.VMEM((R,), jnp.float32),
            pltpu.VMEM((R,), jnp.float32),
        ],
    )
    return sc(act, proj, gamma, ln_beta)


# speedup vs baseline: 1.1219x; 1.1219x over previous
import functools

import jax
import jax.numpy as jnp
from jax import lax
from jax.experimental import pallas as pl
import jax.experimental.pallas.tpu as pltpu
from jax.experimental.pallas import tpu_sc as plsc

_TOP_K = 8
_NEG = -1e30
_L = 16
_RPW = 2


def _tc_body(facts_ref, beta_ref, al_ref, rs_ref, W_ref, ap_ref):
    facts = facts_ref[...]
    B = facts.shape[0]
    beta = beta_ref[...]
    mask = jnp.where(beta > 0.0, 1.0, 0.0)
    mask_bf = mask.astype(jnp.bfloat16)

    dn = (((1,), (1,)), ((), ()))
    mm_bf = functools.partial(jax.lax.dot_general, dimension_numbers=dn,
                              preferred_element_type=jnp.float32)

    f_hi = facts.astype(jnp.bfloat16)
    r1 = facts - f_hi.astype(jnp.float32)
    f_lo = r1.astype(jnp.bfloat16)
    f_lo2 = (r1 - f_lo.astype(jnp.float32)).astype(jnp.bfloat16)

    log_f = jnp.log(jnp.maximum(facts, 1e-30))
    log_1mf = jnp.log(jnp.maximum(1.0 - facts, 1e-30))
    big_lhs = jnp.concatenate(
        [log_f.astype(jnp.bfloat16), log_1mf.astype(jnp.bfloat16),
         f_hi, f_lo, f_lo2], axis=0)
    big = mm_bf(big_lhs, mask_bf)
    prods = jnp.exp(big[:2 * B])
    and_agg = prods[:B]
    or_agg = 1.0 - prods[B:]
    s_sum = big[2 * B:3 * B] + big[3 * B:4 * B] + big[4 * B:]
    cnt = jnp.sum(mask, axis=1)[None, :] + 1e-08
    k_of_n = s_sum / cnt

    w = jax.nn.softmax(al_ref[...].T, axis=0)
    mixed = (w[0][None, :] * and_agg + w[1][None, :] * or_agg
             + w[2][None, :] * k_of_n + w[3][None, :] * (1.0 - k_of_n))
    act = mixed * jax.nn.sigmoid(rs_ref[...])

    W_f32 = W_ref[...]
    w_hi = W_f32.astype(jnp.bfloat16)
    w_lo = (W_f32 - w_hi.astype(jnp.float32)).astype(jnp.bfloat16)
    R = W_f32.shape[0]
    P = mm_bf(jnp.concatenate([f_hi, f_lo], axis=0),
              jnp.concatenate([w_hi, w_lo], axis=0))
    proj = (P[:B, :R] + P[:B, R:]) + (P[B:, :R] + P[B:, R:])
    ap_ref[...] = jnp.stack([act, proj], axis=1)


def _rsqrt16(v):
    i = plsc.bitcast(v, jnp.int32)
    i = jnp.int32(0x5F3759DF) - lax.shift_right_arithmetic(i, jnp.int32(1))
    y = plsc.bitcast(i, jnp.float32)
    half = v * 0.5
    for _ in range(3):
        y = y * (1.5 - half * y * y)
    return y


def _sc_body(ap_hbm, gl_hbm, out_hbm, ap_v, out_v, gl_v, sem1, sem2):
    R = ap_hbm.shape[2]
    nch = R // _L
    wid = lax.axis_index("s") * 2 + lax.axis_index("c")
    base = wid * _RPW
    cp1 = pltpu.make_async_copy(ap_hbm.at[pl.ds(base, _RPW)], ap_v, sem1)
    cp2 = pltpu.make_async_copy(gl_hbm, gl_v, sem2)
    cp1.start()
    cp2.start()
    cp1.wait()
    cp2.wait()

    for r in range(_RPW):
        def p1_step(c, tops):
            x = ap_v[r, 0, pl.ds(c * _L, _L)]
            new = []
            for j in range(_TOP_K):
                new.append(jnp.maximum(tops[j], x))
                x = jnp.minimum(tops[j], x)
            return tuple(new)
        tops = lax.fori_loop(
            0, nch, p1_step,
            tuple(jnp.full((_L,), _NEG, jnp.float32) for _ in range(_TOP_K)))
        tops = list(tops)
        t = jnp.float32(0.0)
        for it in range(_TOP_K):
            m = jnp.max(tops[0])
            for j in range(1, _TOP_K):
                m = jnp.maximum(m, jnp.max(tops[j]))
            t = m
            if it < _TOP_K - 1:
                removed = jnp.int32(0)
                for j in range(_TOP_K):
                    eq = tops[j] == m
                    eqi = eq.astype(jnp.int32)
                    first = jnp.logical_and(eq, lax.cumsum(eqi) == 1)
                    any_eq = jnp.max(eqi) > 0
                    take = jnp.logical_and(any_eq, removed == 0)
                    sel = jnp.logical_and(first, jnp.broadcast_to(take, (_L,)))
                    tops[j] = jnp.where(sel, _NEG, tops[j])
                    removed = removed + take.astype(jnp.int32)
        def p3a_step(c, acc):
            x = ap_v[r, 0, pl.ds(c * _L, _L)]
            return acc + jnp.sum((x > t).astype(jnp.int32))
        gt_count = lax.fori_loop(0, nch, p3a_step, jnp.int32(0))
        need = _TOP_K - gt_count
        def p3b_step(c, carry):
            run_eq, s1, s2 = carry
            x = ap_v[r, 0, pl.ds(c * _L, _L)]
            gt = x > t
            eq = x == t
            eqi = eq.astype(jnp.int32)
            rank = lax.cumsum(eqi) - 1 + run_eq
            gate = jnp.logical_or(gt, jnp.logical_and(eq, rank < need))
            y = ap_v[r, 1, pl.ds(c * _L, _L)] + jnp.where(gate, x, 0.0)
            out_v[r, pl.ds(c * _L, _L)] = y
            return (run_eq + jnp.sum(eqi), s1 + jnp.sum(y),
                    s2 + jnp.sum(y * y))
        run_eq, s1, s2 = lax.fori_loop(
            0, nch, p3b_step, (jnp.int32(0), jnp.float32(0.0),
                               jnp.float32(0.0)))
        inv_r = jnp.float32(1.0 / R)
        mu = s1 * inv_r
        var = s2 * inv_r - mu * mu
        rsq = _rsqrt16(jnp.broadcast_to(var + 1e-05, (_L,)))
        muv = jnp.broadcast_to(mu, (_L,))

        def ln_step(c, carry):
            y = out_v[r, pl.ds(c * _L, _L)]
            g = gl_v[0, pl.ds(c * _L, _L)]
            b = gl_v[1, pl.ds(c * _L, _L)]
            out_v[r, pl.ds(c * _L, _L)] = (y - muv) * rsq * g + b
            return carry
        lax.fori_loop(0, nch, ln_step, jnp.int32(0))

    pltpu.sync_copy(out_v, out_hbm.at[pl.ds(base, _RPW), :])


def kernel(facts, beta, aggregator_logits, rule_strength_raw, W, gamma,
           ln_beta):
    B, _ = facts.shape
    R, _ = beta.shape
    ap = pl.pallas_call(
        _tc_body,
        out_shape=jax.ShapeDtypeStruct((B, 2, R), jnp.float32),
    )(facts, beta, aggregator_logits, rule_strength_raw[None, :], W)

    gl = jnp.stack([gamma, ln_beta], axis=0)
    mesh = plsc.VectorSubcoreMesh(core_axis_name="c", subcore_axis_name="s")
    sc = pl.kernel(
        _sc_body,
        mesh=mesh,
        compiler_params=pltpu.CompilerParams(needs_layout_passes=False),
        out_type=jax.ShapeDtypeStruct((B, R), jnp.float32),
        scratch_types=[
            pltpu.VMEM((_RPW, 2, R), jnp.float32),
            pltpu.VMEM((_RPW, R), jnp.float32),
            pltpu.VMEM((2, R), jnp.float32),
            pltpu.SemaphoreType.DMA,
            pltpu.SemaphoreType.DMA,
        ],
    )
    return sc(ap, gl)


# speedup vs baseline: 4.2039x; 3.7471x over previous
import functools

import jax
import jax.numpy as jnp
from jax.experimental import pallas as pl
import jax.experimental.pallas.tpu as pltpu

_TOP_K = 8
_NEG = -1e30


def _body(facts_ref, beta_ref, al_ref, rs_ref, W_ref, gamma_ref, lnb_ref,
          out_ref, act_ref, pre_ref):
    i = pl.program_id(0)
    facts = facts_ref[...]
    B = facts.shape[0]
    half = beta_ref.shape[0]
    mask = jnp.where(beta_ref[...] > 0.0, 1.0, 0.0)
    mask_bf = mask.astype(jnp.bfloat16)

    dn = (((1,), (1,)), ((), ()))
    mm_bf = functools.partial(jax.lax.dot_general, dimension_numbers=dn,
                              preferred_element_type=jnp.float32)

    f_hi = facts.astype(jnp.bfloat16)
    r1 = facts - f_hi.astype(jnp.float32)
    f_lo = r1.astype(jnp.bfloat16)
    f_lo2 = (r1 - f_lo.astype(jnp.float32)).astype(jnp.bfloat16)

    log_f = jnp.log(jnp.maximum(facts, 1e-30))
    log_1mf = jnp.log(jnp.maximum(1.0 - facts, 1e-30))
    big_lhs = jnp.concatenate(
        [log_f.astype(jnp.bfloat16), log_1mf.astype(jnp.bfloat16),
         f_hi, f_lo, f_lo2], axis=0)
    big = mm_bf(big_lhs, mask_bf)
    prods = jnp.exp(big[:2 * B])
    and_agg = prods[:B]
    or_agg = 1.0 - prods[B:]
    s_sum = big[2 * B:3 * B] + big[3 * B:4 * B] + big[4 * B:]
    cnt = jnp.sum(mask, axis=1)[None, :] + 1e-08
    k_of_n = s_sum / cnt

    w = jax.nn.softmax(al_ref[...].T, axis=0)
    mixed = (w[0][None, :] * and_agg + w[1][None, :] * or_agg
             + w[2][None, :] * k_of_n + w[3][None, :] * (1.0 - k_of_n))
    act_half = mixed * jax.nn.sigmoid(rs_ref[...])

    W_f32 = W_ref[...]
    w_hi = W_f32.astype(jnp.bfloat16)
    w_lo = (W_f32 - w_hi.astype(jnp.float32)).astype(jnp.bfloat16)
    P = mm_bf(jnp.concatenate([f_hi, f_lo], axis=0),
              jnp.concatenate([w_hi, w_lo], axis=0))
    proj_half = (P[:B, :half] + P[:B, half:]) + (P[B:, :half] + P[B:, half:])

    @pl.when(i == 0)
    def _():
        act_ref[:, :half] = act_half
        pre_ref[:, :half] = proj_half

    @pl.when(i == 1)
    def _():
        act_ref[:, half:] = act_half
        pre_ref[:, half:] = proj_half

        act = act_ref[...]
        iota = jax.lax.broadcasted_iota(jnp.int32, act.shape, 1)
        a = act
        gate = jnp.zeros_like(act)
        for _ in range(_TOP_K):
            m = jnp.max(a, axis=1, keepdims=True)
            idx = jnp.min(jnp.where(a == m, iota, act.shape[1]), axis=1,
                          keepdims=True)
            sel = iota == idx
            gate = jnp.where(sel, 1.0, gate)
            a = jnp.where(sel, _NEG, a)

        pre = pre_ref[...] + act * gate
        mu = jnp.mean(pre, axis=1, keepdims=True)
        var = jnp.mean(pre * pre, axis=1, keepdims=True) - mu * mu
        out_ref[...] = ((pre - mu) * jax.lax.rsqrt(var + 1e-05)
                        * gamma_ref[...] + lnb_ref[...])


def kernel(facts, beta, aggregator_logits, rule_strength_raw, W, gamma,
           ln_beta):
    B, D = facts.shape
    R, _ = beta.shape
    half = R // 2
    return pl.pallas_call(
        _body,
        grid=(2,),
        in_specs=[
            pl.BlockSpec((B, D), lambda i: (0, 0)),
            pl.BlockSpec((half, D), lambda i: (i, 0)),
            pl.BlockSpec((half, 4), lambda i: (i, 0)),
            pl.BlockSpec((1, half), lambda i: (0, i)),
            pl.BlockSpec((half, D), lambda i: (i, 0)),
            pl.BlockSpec((1, R), lambda i: (0, 0)),
            pl.BlockSpec((1, R), lambda i: (0, 0)),
        ],
        out_specs=pl.BlockSpec((B, R), lambda i: (0, 0)),
        out_shape=jax.ShapeDtypeStruct((B, R), jnp.float32),
        scratch_shapes=[
            pltpu.VMEM((B, R), jnp.float32),
            pltpu.VMEM((B, R), jnp.float32),
        ],
    )(facts, beta, aggregator_logits, rule_strength_raw[None, :], W,
      gamma[None, :], ln_beta[None, :])


# speedup vs baseline: 4.3113x; 1.0256x over previous
import functools

import jax
import jax.numpy as jnp
from jax.experimental import pallas as pl

_TOP_K = 8
_NEG = -1e30


def _body(facts_ref, beta_ref, al_ref, rs_ref, W_ref, gamma_ref, lnb_ref,
          out_ref):
    facts = facts_ref[...]
    B = facts.shape[0]
    beta = beta_ref[...]
    mask = jnp.where(beta > 0.0, 1.0, 0.0)
    mask_bf = mask.astype(jnp.bfloat16)

    dn = (((1,), (1,)), ((), ()))
    mm_bf = functools.partial(jax.lax.dot_general, dimension_numbers=dn,
                              preferred_element_type=jnp.float32)

    f_hi = facts.astype(jnp.bfloat16)
    r1 = facts - f_hi.astype(jnp.float32)
    f_lo = r1.astype(jnp.bfloat16)
    f_lo2 = (r1 - f_lo.astype(jnp.float32)).astype(jnp.bfloat16)

    log_f = jnp.log(jnp.maximum(facts, 1e-30))
    log_1mf = jnp.log(jnp.maximum(1.0 - facts, 1e-30))
    big_lhs = jnp.concatenate(
        [log_f.astype(jnp.bfloat16), log_1mf.astype(jnp.bfloat16),
         f_hi, f_lo, f_lo2], axis=0)
    big = mm_bf(big_lhs, mask_bf)
    prods = jnp.exp(big[:2 * B])
    and_agg = prods[:B]
    or_agg = 1.0 - prods[B:]
    s_sum = big[2 * B:3 * B] + big[3 * B:4 * B] + big[4 * B:]
    cnt = jnp.sum(mask, axis=1)[None, :] + 1e-08
    k_of_n = s_sum / cnt

    w = jax.nn.softmax(al_ref[...].T, axis=0)
    mixed = (w[0][None, :] * and_agg + w[1][None, :] * or_agg
             + w[2][None, :] * k_of_n + w[3][None, :] * (1.0 - k_of_n))
    act = mixed * jax.nn.sigmoid(rs_ref[...])

    iota = jax.lax.broadcasted_iota(jnp.int32, act.shape, 1)
    a = act
    gate = jnp.zeros_like(act)
    for _ in range(_TOP_K):
        m = jnp.max(a, axis=1, keepdims=True)
        idx = jnp.min(jnp.where(a == m, iota, act.shape[1]), axis=1,
                      keepdims=True)
        sel = iota == idx
        gate = jnp.where(sel, 1.0, gate)
        a = jnp.where(sel, _NEG, a)

    W_f32 = W_ref[...]
    w_hi = W_f32.astype(jnp.bfloat16)
    w_lo = (W_f32 - w_hi.astype(jnp.float32)).astype(jnp.bfloat16)
    R = W_f32.shape[0]
    P = mm_bf(jnp.concatenate([f_hi, f_lo], axis=0),
              jnp.concatenate([w_hi, w_lo], axis=0))
    proj = (P[:B, :R] + P[:B, R:]) + (P[B:, :R] + P[B:, R:])
    pre = proj + act * gate
    mu = jnp.mean(pre, axis=1, keepdims=True)
    var = jnp.mean(pre * pre, axis=1, keepdims=True) - mu * mu
    out_ref[...] = ((pre - mu) * jax.lax.rsqrt(var + 1e-05)
                    * gamma_ref[...] + lnb_ref[...])


def kernel(facts, beta, aggregator_logits, rule_strength_raw, W, gamma,
           ln_beta):
    B, _ = facts.shape
    R, _ = beta.shape
    return pl.pallas_call(
        _body,
        out_shape=jax.ShapeDtypeStruct((B, R), jnp.float32),
    )(facts, beta, aggregator_logits, rule_strength_raw[None, :], W,
      gamma[None, :], ln_beta[None, :])
